# baseline (speedup 1.0000x reference)
import functools

import jax
import jax.numpy as jnp
from jax import lax
from jax.experimental import pallas as pl
from jax.experimental.pallas import tpu as pltpu
from jax.experimental.pallas import tpu_sc as plsc

B = 4096
F = 26
V = 100000
C = 64
EPS = 1e-5

_info = plsc.get_sparse_core_info()
NC = _info.num_cores
NS = _info.num_subcores
L = _info.num_lanes
NW = NC * NS
BW = B // NW
CL = C // L


def _rsqrt(x):
    i = lax.bitcast_convert_type(x, jnp.int32)
    i = jnp.int32(0x5F3759DF) - lax.shift_right_logical(i, 1)
    y = lax.bitcast_convert_type(i, jnp.float32)
    for _ in range(3):
        y = y * (1.5 - 0.5 * x * y * y)
    return y


@functools.partial(
    pl.kernel,
    out_type=jax.ShapeDtypeStruct((B, C), jnp.float32),
    mesh=plsc.VectorSubcoreMesh(core_axis_name="c", subcore_axis_name="s"),
    compiler_params=pltpu.CompilerParams(needs_layout_passes=False,
                                         use_tc_tiling_on_sc=False),
    scratch_types=[
        pltpu.VMEM((F * BW,), jnp.int32),
        pltpu.VMEM((BW,), jnp.int32),
        pltpu.VMEM((BW, C), jnp.float32),
        pltpu.VMEM((BW, C), jnp.float32),
        pltpu.VMEM((4 * C,), jnp.float32),
        pltpu.SemaphoreType.DMA,
    ],
)
def _sc_embed(tables_hbm, temporal_hbm, xt_hbm, t_hbm, ng_hbm, nb_hbm,
              sg_hbm, sb_hbm, out_hbm,
              idx_v, tw_v, gbuf_v, acc_v, prm_v, sem0):
    cid = lax.axis_index("c")
    sid = lax.axis_index("s")
    wid = sid * NC + cid
    base = wid * BW

    pltpu.sync_copy(t_hbm.at[pl.ds(base, BW)], tw_v)
    tcopy = pltpu.async_copy(temporal_hbm.at[tw_v], acc_v, sem0)

    pltpu.sync_copy(ng_hbm, prm_v.at[pl.ds(0, C)])
    pltpu.sync_copy(nb_hbm, prm_v.at[pl.ds(C, C)])
    pltpu.sync_copy(sg_hbm, prm_v.at[pl.ds(2 * C, C)])
    pltpu.sync_copy(sb_hbm, prm_v.at[pl.ds(3 * C, C)])

    def build_idx(f, _):
        pltpu.sync_copy(xt_hbm.at[pl.ds(f * B + base, BW)],
                        idx_v.at[pl.ds(f * BW, BW)])
        return 0
    lax.fori_loop(0, F, build_idx, 0)

    gvec = [prm_v[pl.ds(u * L, L)] for u in range(CL)]
    bvec = [prm_v[pl.ds(C + u * L, L)] for u in range(CL)]
    sgvec = [prm_v[pl.ds(2 * C + u * L, L)] for u in range(CL)]
    sbvec = [prm_v[pl.ds(3 * C + u * L, L)] for u in range(CL)]

    tcopy.wait()

    def row_ln_update(b, src_ref, gain):
        v = [src_ref[b, pl.ds(u * L, L)] for u in range(CL)]
        sv = (v[0] + v[1]) + (v[2] + v[3])
        qv = (v[0] * v[0] + v[1] * v[1]) + (v[2] * v[2] + v[3] * v[3])
        tot = jnp.sum(sv)
        totq = jnp.sum(qv)
        mu = tot * (1.0 / C)
        var = totq * (1.0 / C) - mu * mu
        muv = jnp.full((L,), mu, jnp.float32)
        rv = _rsqrt(jnp.full((L,), var + EPS, jnp.float32))
        return [gain[u] * ((v[u] - muv) * rv) for u in range(CL)]

    def field_body(f, _):
        pltpu.async_copy(tables_hbm.at[f].at[idx_v.at[pl.ds(f * BW, BW)]],
                         gbuf_v, sem0).wait()

        def row_body(b, _):
            upd = row_ln_update(b, gbuf_v, gvec)
            for u in range(CL):
                p = pl.ds(u * L, L)
                acc_v[b, p] = acc_v[b, p] + upd[u]
            return 0
        lax.fori_loop(0, BW, row_body, 0)
        return 0
    lax.fori_loop(0, F, field_body, 0)

    def tail_body(b, _):
        for u in range(CL):
            p = pl.ds(u * L, L)
            acc_v[b, p] = acc_v[b, p] + float(F) * bvec[u]
        upd = row_ln_update(b, acc_v, sgvec)
        for u in range(CL):
            acc_v[b, pl.ds(u * L, L)] = upd[u] + sbvec[u]
        return 0
    lax.fori_loop(0, BW, tail_body, 0)

    pltpu.sync_copy(acc_v, out_hbm.at[pl.ds(base, BW)])


def kernel(x, t, pad, tables, temporal_table, norm_g, norm_b, sum_g, sum_b):
    xt = x.T.reshape(F * B)
    out = _sc_embed(tables, temporal_table, xt, t,
                    norm_g, norm_b, sum_g, sum_b)
    return (out, t, pad)


# speedup vs baseline: 2.2514x; 2.2514x over previous
import functools

import jax
import jax.numpy as jnp
from jax import lax
from jax.experimental import pallas as pl
from jax.experimental.pallas import tpu as pltpu
from jax.experimental.pallas import tpu_sc as plsc

B = 4096
F = 26
V = 100000
C = 64
T = 512
EPS = 1e-5

_info = plsc.get_sparse_core_info()
NC = _info.num_cores
NS = _info.num_subcores
L = _info.num_lanes
NW = NC * NS
BW = B // NW
CL = C // L
CH = 32
NCH = (F * BW) // CH
TCH = BW // CH


def _rsqrt(x):
    i = lax.bitcast_convert_type(x, jnp.int32)
    i = jnp.int32(0x5F3759DF) - lax.shift_right_logical(i, 1)
    y = lax.bitcast_convert_type(i, jnp.float32)
    for _ in range(3):
        y = y * (1.5 - 0.5 * x * y * y)
    return y


@functools.partial(
    pl.kernel,
    out_type=jax.ShapeDtypeStruct((B * C,), jnp.float32),
    mesh=plsc.VectorSubcoreMesh(core_axis_name="c", subcore_axis_name="s"),
    compiler_params=pltpu.CompilerParams(needs_layout_passes=False,
                                         use_tc_tiling_on_sc=True),
    scratch_types=[
        pltpu.VMEM((F * BW,), jnp.int32),
        pltpu.VMEM((BW,), jnp.int32),
        pltpu.VMEM((CH, C), jnp.float32),
        pltpu.VMEM((CH, C), jnp.float32),
        pltpu.VMEM((BW * C,), jnp.float32),
        pltpu.VMEM((4 * C,), jnp.float32),
        pltpu.SemaphoreType.DMA,
        pltpu.SemaphoreType.DMA,
    ],
)
def _sc_embed(tables_hbm, temporal_hbm, xt_hbm, t_hbm, ng_hbm, nb_hbm,
              sg_hbm, sb_hbm, out_hbm,
              idx_v, tw_v, bufa_v, bufb_v, acc_v, prm_v, sema, semb):
    cid = lax.axis_index("c")
    sid = lax.axis_index("s")
    wid = sid * NC + cid
    base = wid * BW

    pltpu.sync_copy(t_hbm.at[pl.ds(base, BW)], tw_v)

    pltpu.sync_copy(ng_hbm, prm_v.at[pl.ds(0, C)])
    pltpu.sync_copy(nb_hbm, prm_v.at[pl.ds(C, C)])
    pltpu.sync_copy(sg_hbm, prm_v.at[pl.ds(2 * C, C)])
    pltpu.sync_copy(sb_hbm, prm_v.at[pl.ds(3 * C, C)])

    def build_idx(f, _):
        pltpu.sync_copy(xt_hbm.at[pl.ds(f * B + base, BW)],
                        idx_v.at[pl.ds(f * BW, BW)])
        off = f * V
        for j in range(BW // L):
            p = pl.ds(f * BW + j * L, L)
            idx_v[p] = idx_v[p] + off
        return 0
    lax.fori_loop(0, F, build_idx, 0)

    zero = jnp.zeros((L,), jnp.float32)
    def zacc(j, _):
        acc_v[pl.ds(j * L, L)] = zero
        return 0
    lax.fori_loop(0, (BW * C) // L, zacc, 0)

    gvec = [prm_v[pl.ds(u * L, L)] for u in range(CL)]
    bvec = [prm_v[pl.ds(C + u * L, L)] for u in range(CL)]
    sgvec = [prm_v[pl.ds(2 * C + u * L, L)] for u in range(CL)]
    sbvec = [prm_v[pl.ds(3 * C + u * L, L)] for u in range(CL)]

    def fire(src_hbm, q_ref, qpos, buf, sem):
        for g in range(CH // L):
            qv = q_ref[pl.ds(qpos + g * L, L)]
            for u in range(L):
                pltpu.async_copy(src_hbm.at[qv[u]], buf.at[g * L + u], sem)

    def drain(buf, sem):
        pltpu.make_async_copy(tables_hbm.at[pl.ds(0, CH)], buf, sem).wait()

    def ln_pieces(v):
        sv = (v[0] + v[1]) + (v[2] + v[3])
        qv = (v[0] * v[0] + v[1] * v[1]) + (v[2] * v[2] + v[3] * v[3])
        mu = jnp.sum(sv) * (1.0 / C)
        var = jnp.sum(qv) * (1.0 / C) - mu * mu
        muv = jnp.full((L,), mu, jnp.float32)
        rv = _rsqrt(jnp.full((L,), var + EPS, jnp.float32))
        return muv, rv

    def compute_chunk(k, buf):
        arow0 = lax.bitwise_and(k, TCH - 1) * CH

        def row_body(j, _):
            v = [buf[j, pl.ds(u * L, L)] for u in range(CL)]
            muv, rv = ln_pieces(v)
            for u in range(CL):
                p = pl.ds((arow0 + j) * C + u * L, L)
                acc_v[p] = acc_v[p] + gvec[u] * ((v[u] - muv) * rv)
            return 0
        lax.fori_loop(0, CH, row_body, 0)

    fire(tables_hbm, idx_v, 0, bufa_v, sema)

    def pair_body(i, _):
        ka = 2 * i
        fire(tables_hbm, idx_v, (ka + 1) * CH, bufb_v, semb)
        drain(bufa_v, sema)
        compute_chunk(ka, bufa_v)

        @pl.when(i < NCH // 2 - 1)
        def _():
            fire(tables_hbm, idx_v, (ka + 2) * CH, bufa_v, sema)
        drain(bufb_v, semb)
        compute_chunk(ka + 1, bufb_v)
        return 0
    lax.fori_loop(0, NCH // 2, pair_body, 0)

    fire(temporal_hbm, tw_v, 0, bufa_v, sema)
    for kc in range(TCH):
        if kc + 1 < TCH:
            buf, sem = (bufb_v, semb) if kc % 2 == 0 else (bufa_v, sema)
            fire(temporal_hbm, tw_v, (kc + 1) * CH, buf, sem)
        buf, sem = (bufa_v, sema) if kc % 2 == 0 else (bufb_v, semb)
        drain(buf, sem)

        def tail_body(j, _):
            h = [acc_v[pl.ds((kc * CH + j) * C + u * L, L)]
                 + buf[j, pl.ds(u * L, L)] + float(F) * bvec[u]
                 for u in range(CL)]
            muv, rv = ln_pieces(h)
            for u in range(CL):
                acc_v[pl.ds((kc * CH + j) * C + u * L, L)] = \
                    ((h[u] - muv) * rv) * sgvec[u] + sbvec[u]
            return 0
        lax.fori_loop(0, CH, tail_body, 0)

    pltpu.sync_copy(acc_v, out_hbm.at[pl.ds(base * C, BW * C)])


def kernel(x, t, pad, tables, temporal_table, norm_g, norm_b, sum_g, sum_b):
    tq = tables.reshape(F * V, C)
    xt = x.T.reshape(F * B)
    out = _sc_embed(tq, temporal_table, xt, t, norm_g, norm_b, sum_g, sum_b)
    return (out.reshape(B, C), t, pad)
